# Initial kernel scaffold; baseline (speedup 1.0000x reference)
import jax
import jax.numpy as jnp
from jax.experimental import pallas as pl


def kernel(x, edge_index, edge_weight, batch, sigma, Wg, bg, Wrel0, brel0, Wroot0, Wrel1, brel1, Wroot1, Wrel2, brel2, Wroot2, W1, b1, W2, b2):
    raise NotImplementedError("write your pallas kernel here")


# speedup vs baseline: 82.0148x; 82.0148x over previous
import functools

import jax
import jax.numpy as jnp
from jax.experimental import pallas as pl

B = 64
NPW = 64
D = 128
H = 128
OUT = 16
WPP = 8

_HI = jax.lax.Precision.HIGHEST


def _dot(a, b):
    return jax.lax.dot_general(a, b, (((1,), (0,)), ((), ())),
                               precision=_HI, preferred_element_type=jnp.float32)


def _dot_t(a, b):
    return jax.lax.dot_general(a, b, (((1,), (1,)), ((), ())),
                               precision=_HI, preferred_element_type=jnp.float32)


def _gnn_kernel(x_ref, inv_ref, wg_ref, bg_ref,
                wrel0_ref, brel0_ref, wroot0_ref,
                wrel1_ref, brel1_ref, wroot1_ref,
                wrel2_ref, brel2_ref, wroot2_ref,
                w1_ref, b1_ref, w2_ref, b2_ref, out_ref):
    inv = inv_ref[0, 0]
    xw = _dot(x_ref[:], wg_ref[:]) + bg_ref[:]

    sims = []
    for w in range(WPP):
        xi = xw[w * NPW:(w + 1) * NPW, :]
        sq = jnp.sum(xi * xi, axis=1, keepdims=True)
        g = _dot_t(xi, xi)
        dist = sq + sq.T - 2.0 * g
        sims.append(jnp.exp(dist * (-inv)))

    h = xw
    layers = ((wrel0_ref, brel0_ref, wroot0_ref),
              (wrel1_ref, brel1_ref, wroot1_ref),
              (wrel2_ref, brel2_ref, wroot2_ref))
    for li, (wrel, brel, wroot) in enumerate(layers):
        sh = jnp.concatenate(
            [_dot(sims[w], h[w * NPW:(w + 1) * NPW, :]) for w in range(WPP)],
            axis=0)
        agg = sh - h
        h_new = _dot(agg, wrel[:]) + brel[:] + _dot(h, wroot[:])
        h = jnp.maximum(h_new, 0.0) if li < 2 else h_new

    pooled = jnp.concatenate(
        [jnp.sum(h[w * NPW:(w + 1) * NPW, :], axis=0, keepdims=True)
         for w in range(WPP)], axis=0) * (1.0 / NPW)
    z = jnp.maximum(_dot(pooled, w1_ref[:]) + b1_ref[:], 0.0)
    out_ref[:] = _dot(z, w2_ref[:]) + b2_ref[:]


@functools.partial(jax.jit, static_argnames=("interpret",))
def _run(x, inv, WgT, bg2, Wrel0T, brel02, Wroot0T, Wrel1T, brel12, Wroot1T,
         Wrel2T, brel22, Wroot2T, W1T, b12, W2T, b22, interpret=False):
    full2d = lambda shape: pl.BlockSpec(shape, lambda i: (0, 0))
    grid = B // WPP
    return pl.pallas_call(
        _gnn_kernel,
        grid=(grid,),
        in_specs=[
            pl.BlockSpec((WPP * NPW, D), lambda i: (i, 0)),
            full2d((1, 1)),
            full2d((D, D)), full2d((1, D)),
            full2d((D, H)), full2d((1, H)), full2d((D, H)),
            full2d((H, H)), full2d((1, H)), full2d((H, H)),
            full2d((H, H)), full2d((1, H)), full2d((H, H)),
            full2d((H, H)), full2d((1, H)),
            full2d((H, OUT)), full2d((1, OUT)),
        ],
        out_specs=pl.BlockSpec((WPP, OUT), lambda i: (i, 0)),
        out_shape=jax.ShapeDtypeStruct((B, OUT), jnp.float32),
        interpret=interpret,
    )(x, inv, WgT, bg2, Wrel0T, brel02, Wroot0T, Wrel1T, brel12, Wroot1T,
      Wrel2T, brel22, Wroot2T, W1T, b12, W2T, b22)


def kernel(x, edge_index, edge_weight, batch, sigma, Wg, bg,
           Wrel0, brel0, Wroot0, Wrel1, brel1, Wroot1, Wrel2, brel2, Wroot2,
           W1, b1, W2, b2):
    inv = (1.0 / (2.0 * sigma[0] ** 2)).reshape(1, 1).astype(jnp.float32)
    return _run(x, inv, Wg.T, bg.reshape(1, D),
                Wrel0.T, brel0.reshape(1, H), Wroot0.T,
                Wrel1.T, brel1.reshape(1, H), Wroot1.T,
                Wrel2.T, brel2.reshape(1, H), Wroot2.T,
                W1.T, b1.reshape(1, H), W2.T, b2.reshape(1, OUT))


# speedup vs baseline: 258.4690x; 3.1515x over previous
import functools

import jax
import jax.numpy as jnp
from jax.experimental import pallas as pl

B = 64
NPW = 64
D = 128
H = 128
OUT = 16
WPP = 8

_HI = jax.lax.Precision.HIGHEST
_LO = jax.lax.Precision.DEFAULT


def _dot(a, b, precision=_LO):
    return jax.lax.dot_general(a, b, (((1,), (0,)), ((), ())),
                               precision=precision,
                               preferred_element_type=jnp.float32)


def _dot_t(a, b, precision=_LO):
    return jax.lax.dot_general(a, b, (((1,), (1,)), ((), ())),
                               precision=precision,
                               preferred_element_type=jnp.float32)


def _gnn_kernel(x_ref, sig_ref, wg_ref, bg_ref,
                wrel0_ref, brel0_ref, wroot0_ref,
                wrel1_ref, brel1_ref, wroot1_ref,
                wrel2_ref, brel2_ref, wroot2_ref,
                w1_ref, b1_ref, w2_ref, b2_ref, out_ref):
    s = sig_ref[0, 0]
    inv = 1.0 / (2.0 * s * s)
    xw = _dot_t(x_ref[:], wg_ref[:]) + bg_ref[:]

    sims = []
    for w in range(WPP):
        xi = xw[w * NPW:(w + 1) * NPW, :]
        sq = jnp.sum(xi * xi, axis=1, keepdims=True)
        g = _dot_t(xi, xi, _HI)
        dist = sq + sq.T - 2.0 * g
        sims.append(jnp.exp(dist * (-inv)))

    h = xw
    layers = ((wrel0_ref, brel0_ref, wroot0_ref),
              (wrel1_ref, brel1_ref, wroot1_ref),
              (wrel2_ref, brel2_ref, wroot2_ref))
    for li, (wrel, brel, wroot) in enumerate(layers):
        sh = jnp.concatenate(
            [_dot(sims[w], h[w * NPW:(w + 1) * NPW, :]) for w in range(WPP)],
            axis=0)
        agg = sh - h
        h_new = _dot_t(agg, wrel[:]) + brel[:] + _dot_t(h, wroot[:])
        h = jnp.maximum(h_new, 0.0) if li < 2 else h_new

    pooled = jnp.concatenate(
        [jnp.sum(h[w * NPW:(w + 1) * NPW, :], axis=0, keepdims=True)
         for w in range(WPP)], axis=0) * (1.0 / NPW)
    z = jnp.maximum(_dot_t(pooled, w1_ref[:]) + b1_ref[:], 0.0)
    out_ref[:] = _dot_t(z, w2_ref[:]) + b2_ref[:]


@functools.partial(jax.jit, static_argnames=("interpret",))
def _run(x, sig, Wg, bg2, Wrel0, brel02, Wroot0, Wrel1, brel12, Wroot1,
         Wrel2, brel22, Wroot2, W1, b12, W2, b22, interpret=False):
    full2d = lambda shape: pl.BlockSpec(shape, lambda i: (0, 0))
    grid = B // WPP
    return pl.pallas_call(
        _gnn_kernel,
        grid=(grid,),
        in_specs=[
            pl.BlockSpec((WPP * NPW, D), lambda i: (i, 0)),
            full2d((1, 1)),
            full2d((D, D)), full2d((1, D)),
            full2d((H, D)), full2d((1, H)), full2d((H, D)),
            full2d((H, H)), full2d((1, H)), full2d((H, H)),
            full2d((H, H)), full2d((1, H)), full2d((H, H)),
            full2d((H, H)), full2d((1, H)),
            full2d((OUT, H)), full2d((1, OUT)),
        ],
        out_specs=pl.BlockSpec((WPP, OUT), lambda i: (i, 0)),
        out_shape=jax.ShapeDtypeStruct((B, OUT), jnp.float32),
        interpret=interpret,
    )(x, sig, Wg, bg2, Wrel0, brel02, Wroot0, Wrel1, brel12, Wroot1,
      Wrel2, brel22, Wroot2, W1, b12, W2, b22)


def kernel(x, edge_index, edge_weight, batch, sigma, Wg, bg,
           Wrel0, brel0, Wroot0, Wrel1, brel1, Wroot1, Wrel2, brel2, Wroot2,
           W1, b1, W2, b2):
    return _run(x, sigma.reshape(1, 1), Wg, bg.reshape(1, D),
                Wrel0, brel0.reshape(1, H), Wroot0,
                Wrel1, brel1.reshape(1, H), Wroot1,
                Wrel2, brel2.reshape(1, H), Wroot2,
                W1, b1.reshape(1, H), W2, b2.reshape(1, OUT))


# speedup vs baseline: 343.6261x; 1.3295x over previous
import functools

import jax
import jax.numpy as jnp
from jax.experimental import pallas as pl

B = 64
NPW = 64
D = 128
H = 128
OUT = 16
WPP = 16
NPP = WPP * NPW

_HI = jax.lax.Precision.HIGHEST
_LO = jax.lax.Precision.DEFAULT


def _dot(a, b, precision=_LO):
    return jax.lax.dot_general(a, b, (((1,), (0,)), ((), ())),
                               precision=precision,
                               preferred_element_type=jnp.float32)


def _dot_t(a, b, precision=_LO):
    return jax.lax.dot_general(a, b, (((1,), (1,)), ((), ())),
                               precision=precision,
                               preferred_element_type=jnp.float32)


def _gnn_kernel(x_ref, sig_ref, wg_ref, bg_ref,
                wrel0_ref, brel0_ref, wroot0_ref,
                wrel1_ref, brel1_ref, wroot1_ref,
                wrel2_ref, brel2_ref, wroot2_ref,
                w1_ref, b1_ref, w2_ref, b2_ref, out_ref):
    s = sig_ref[0, 0]
    inv = 1.0 / (2.0 * s * s)
    xw = _dot_t(x_ref[:], wg_ref[:]) + bg_ref[:]

    sims = []
    for w in range(WPP):
        xi = xw[w * NPW:(w + 1) * NPW, :]
        sq = jnp.sum(xi * xi, axis=1, keepdims=True)
        g = _dot_t(xi, xi, _HI)
        dist = sq + sq.T - 2.0 * g
        sims.append(jnp.exp(dist * (-inv)))

    h = xw
    layers = ((wrel0_ref, brel0_ref, wroot0_ref),
              (wrel1_ref, brel1_ref, wroot1_ref),
              (wrel2_ref, brel2_ref, wroot2_ref))
    for li, (wrel, brel, wroot) in enumerate(layers):
        sh = jnp.concatenate(
            [_dot(sims[w], h[w * NPW:(w + 1) * NPW, :]) for w in range(WPP)],
            axis=0)
        cat = jnp.concatenate([sh, h], axis=1)
        wcat = jnp.concatenate([wrel[:], wroot[:] - wrel[:]], axis=1)
        h_new = _dot_t(cat, wcat) + brel[:]
        h = jnp.maximum(h_new, 0.0) if li < 2 else h_new

    row = jax.lax.broadcasted_iota(jnp.int32, (WPP, NPP), 0)
    col = jax.lax.broadcasted_iota(jnp.int32, (WPP, NPP), 1)
    pmat = jnp.where(col // NPW == row, 1.0 / NPW, 0.0)
    pooled = _dot(pmat, h)
    z = jnp.maximum(_dot_t(pooled, w1_ref[:]) + b1_ref[:], 0.0)
    out_ref[:] = _dot_t(z, w2_ref[:]) + b2_ref[:]


@functools.partial(jax.jit, static_argnames=("interpret",))
def _run(x, sig, Wg, bg2, Wrel0, brel02, Wroot0, Wrel1, brel12, Wroot1,
         Wrel2, brel22, Wroot2, W1, b12, W2, b22, interpret=False):
    full2d = lambda shape: pl.BlockSpec(shape, lambda i: (0, 0))
    grid = B // WPP
    return pl.pallas_call(
        _gnn_kernel,
        grid=(grid,),
        in_specs=[
            pl.BlockSpec((NPP, D), lambda i: (i, 0)),
            full2d((1, 1)),
            full2d((D, D)), full2d((1, D)),
            full2d((H, D)), full2d((1, H)), full2d((H, D)),
            full2d((H, H)), full2d((1, H)), full2d((H, H)),
            full2d((H, H)), full2d((1, H)), full2d((H, H)),
            full2d((H, H)), full2d((1, H)),
            full2d((OUT, H)), full2d((1, OUT)),
        ],
        out_specs=pl.BlockSpec((WPP, OUT), lambda i: (i, 0)),
        out_shape=jax.ShapeDtypeStruct((B, OUT), jnp.float32),
        interpret=interpret,
    )(x, sig, Wg, bg2, Wrel0, brel02, Wroot0, Wrel1, brel12, Wroot1,
      Wrel2, brel22, Wroot2, W1, b12, W2, b22)


def kernel(x, edge_index, edge_weight, batch, sigma, Wg, bg,
           Wrel0, brel0, Wroot0, Wrel1, brel1, Wroot1, Wrel2, brel2, Wroot2,
           W1, b1, W2, b2):
    return _run(x, sigma.reshape(1, 1), Wg, bg.reshape(1, D),
                Wrel0, brel0.reshape(1, H), Wroot0,
                Wrel1, brel1.reshape(1, H), Wroot1,
                Wrel2, brel2.reshape(1, H), Wroot2,
                W1, b1.reshape(1, H), W2, b2.reshape(1, OUT))


# speedup vs baseline: 344.2598x; 1.0018x over previous
import functools

import jax
import jax.numpy as jnp
from jax.experimental import pallas as pl
from jax.experimental.pallas import tpu as pltpu

B = 64
NPW = 64
D = 128
H = 128
OUT = 16
WPP = 16
NPP = WPP * NPW

_HI = jax.lax.Precision.HIGHEST
_LO = jax.lax.Precision.DEFAULT


def _dot(a, b, precision=_LO):
    return jax.lax.dot_general(a, b, (((1,), (0,)), ((), ())),
                               precision=precision,
                               preferred_element_type=jnp.float32)


def _dot_t(a, b, precision=_LO):
    return jax.lax.dot_general(a, b, (((1,), (1,)), ((), ())),
                               precision=precision,
                               preferred_element_type=jnp.float32)


def _gnn_kernel(x_ref, sig_ref, wg_ref, bg_ref,
                wrel0_ref, brel0_ref, wroot0_ref,
                wrel1_ref, brel1_ref, wroot1_ref,
                wrel2_ref, brel2_ref, wroot2_ref,
                w1_ref, b1_ref, w2_ref, b2_ref, out_ref):
    s = sig_ref[0, 0]
    inv = 1.0 / (2.0 * s * s)
    xw = _dot_t(x_ref[:], wg_ref[:]) + bg_ref[:]

    sims = []
    for w in range(WPP):
        xi = xw[w * NPW:(w + 1) * NPW, :]
        sq = jnp.sum(xi * xi, axis=1, keepdims=True)
        g = _dot_t(xi, xi, _HI)
        dist = sq + sq.T - 2.0 * g
        sims.append(jnp.exp(dist * (-inv)))

    h = xw
    layers = ((wrel0_ref, brel0_ref, wroot0_ref),
              (wrel1_ref, brel1_ref, wroot1_ref),
              (wrel2_ref, brel2_ref, wroot2_ref))
    for li, (wrel, brel, wroot) in enumerate(layers):
        sh = jnp.concatenate(
            [_dot(sims[w], h[w * NPW:(w + 1) * NPW, :]) for w in range(WPP)],
            axis=0)
        cat = jnp.concatenate([sh, h], axis=1)
        wcat = jnp.concatenate([wrel[:], wroot[:] - wrel[:]], axis=1)
        h_new = _dot_t(cat, wcat) + brel[:]
        h = jnp.maximum(h_new, 0.0) if li < 2 else h_new

    row = jax.lax.broadcasted_iota(jnp.int32, (WPP, NPP), 0)
    col = jax.lax.broadcasted_iota(jnp.int32, (WPP, NPP), 1)
    pmat = jnp.where(col // NPW == row, 1.0 / NPW, 0.0)
    pooled = _dot(pmat, h)
    z = jnp.maximum(_dot_t(pooled, w1_ref[:]) + b1_ref[:], 0.0)
    out_ref[:] = _dot_t(z, w2_ref[:]) + b2_ref[:]


@functools.partial(jax.jit, static_argnames=("interpret",))
def _run(x, sig, Wg, bg2, Wrel0, brel02, Wroot0, Wrel1, brel12, Wroot1,
         Wrel2, brel22, Wroot2, W1, b12, W2, b22, interpret=False):
    full2d = lambda shape: pl.BlockSpec(shape, lambda i: (0, 0))
    grid = B // WPP
    return pl.pallas_call(
        _gnn_kernel,
        grid=(grid,),
        in_specs=[
            pl.BlockSpec((NPP, D), lambda i: (i, 0)),
            full2d((1, 1)),
            full2d((D, D)), full2d((1, D)),
            full2d((H, D)), full2d((1, H)), full2d((H, D)),
            full2d((H, H)), full2d((1, H)), full2d((H, H)),
            full2d((H, H)), full2d((1, H)), full2d((H, H)),
            full2d((H, H)), full2d((1, H)),
            full2d((OUT, H)), full2d((1, OUT)),
        ],
        out_specs=pl.BlockSpec((WPP, OUT), lambda i: (i, 0)),
        out_shape=jax.ShapeDtypeStruct((B, OUT), jnp.float32),
        compiler_params=pltpu.CompilerParams(
            dimension_semantics=("parallel",)),
        interpret=interpret,
    )(x, sig, Wg, bg2, Wrel0, brel02, Wroot0, Wrel1, brel12, Wroot1,
      Wrel2, brel22, Wroot2, W1, b12, W2, b22)


def kernel(x, edge_index, edge_weight, batch, sigma, Wg, bg,
           Wrel0, brel0, Wroot0, Wrel1, brel1, Wroot1, Wrel2, brel2, Wroot2,
           W1, b1, W2, b2):
    return _run(x, sigma.reshape(1, 1), Wg, bg.reshape(1, D),
                Wrel0, brel0.reshape(1, H), Wroot0,
                Wrel1, brel1.reshape(1, H), Wroot1,
                Wrel2, brel2.reshape(1, H), Wroot2,
                W1, b1.reshape(1, H), W2, b2.reshape(1, OUT))


# speedup vs baseline: 355.7772x; 1.0335x over previous
import functools

import jax
import jax.numpy as jnp
from jax.experimental import pallas as pl
from jax.experimental.pallas import tpu as pltpu

B = 64
NPW = 64
D = 128
H = 128
OUT = 16
WPP = 16
NPP = WPP * NPW

_HI = jax.lax.Precision.HIGHEST
_LO = jax.lax.Precision.DEFAULT


def _dot(a, b, precision=_LO):
    return jax.lax.dot_general(a, b, (((1,), (0,)), ((), ())),
                               precision=precision,
                               preferred_element_type=jnp.float32)


def _dot_t(a, b, precision=_LO):
    return jax.lax.dot_general(a, b, (((1,), (1,)), ((), ())),
                               precision=precision,
                               preferred_element_type=jnp.float32)


def _gnn_kernel(x_ref, sig_ref, wg_ref, bg_ref,
                wrel0_ref, brel0_ref, wroot0_ref,
                wrel1_ref, brel1_ref, wroot1_ref,
                wrel2_ref, brel2_ref, wroot2_ref,
                w1_ref, b1_ref, w2_ref, b2_ref, out_ref):
    s = sig_ref[0, 0]
    inv = 1.0 / (2.0 * s * s)
    xw = _dot_t(x_ref[:], wg_ref[:]) + bg_ref[:]

    sims = []
    for w in range(WPP):
        xi = xw[w * NPW:(w + 1) * NPW, :]
        sq = jnp.sum(xi * xi, axis=1, keepdims=True)
        g = _dot_t(xi, xi, _HI)
        dist = sq + sq.T - 2.0 * g
        sims.append(jnp.exp(dist * (-inv)))

    h = xw
    for wrel, brel, wroot in ((wrel0_ref, brel0_ref, wroot0_ref),
                              (wrel1_ref, brel1_ref, wroot1_ref)):
        sh = jnp.concatenate(
            [_dot(sims[w], h[w * NPW:(w + 1) * NPW, :]) for w in range(WPP)],
            axis=0)
        cat = jnp.concatenate([sh, h], axis=1)
        wcat = jnp.concatenate([wrel[:], wroot[:] - wrel[:]], axis=1)
        h = jnp.maximum(_dot_t(cat, wcat) + brel[:], 0.0)

    row = jax.lax.broadcasted_iota(jnp.int32, (WPP, NPP), 0)
    col = jax.lax.broadcasted_iota(jnp.int32, (WPP, NPP), 1)
    blockmask = col // NPW == row
    csbig = jnp.concatenate(
        [jnp.sum(sims[w], axis=0, keepdims=True) for w in range(WPP)],
        axis=1)
    csmat = jnp.where(blockmask, csbig * (1.0 / NPW), 0.0)
    pmat = jnp.where(blockmask, 1.0 / NPW, 0.0)
    both = _dot(jnp.concatenate([csmat, pmat], axis=0), h)
    psh, ph = both[:WPP], both[WPP:]
    cat2 = jnp.concatenate([psh, ph], axis=1)
    wcat2 = jnp.concatenate([wrel2_ref[:], wroot2_ref[:] - wrel2_ref[:]],
                            axis=1)
    pooled = _dot_t(cat2, wcat2) + brel2_ref[:]
    z = jnp.maximum(_dot_t(pooled, w1_ref[:]) + b1_ref[:], 0.0)
    out_ref[:] = _dot_t(z, w2_ref[:]) + b2_ref[:]


@functools.partial(jax.jit, static_argnames=("interpret",))
def _run(x, sig, Wg, bg2, Wrel0, brel02, Wroot0, Wrel1, brel12, Wroot1,
         Wrel2, brel22, Wroot2, W1, b12, W2, b22, interpret=False):
    full2d = lambda shape: pl.BlockSpec(shape, lambda i: (0, 0))
    grid = B // WPP
    return pl.pallas_call(
        _gnn_kernel,
        grid=(grid,),
        in_specs=[
            pl.BlockSpec((NPP, D), lambda i: (i, 0)),
            full2d((1, 1)),
            full2d((D, D)), full2d((1, D)),
            full2d((H, D)), full2d((1, H)), full2d((H, D)),
            full2d((H, H)), full2d((1, H)), full2d((H, H)),
            full2d((H, H)), full2d((1, H)), full2d((H, H)),
            full2d((H, H)), full2d((1, H)),
            full2d((OUT, H)), full2d((1, OUT)),
        ],
        out_specs=pl.BlockSpec((WPP, OUT), lambda i: (i, 0)),
        out_shape=jax.ShapeDtypeStruct((B, OUT), jnp.float32),
        compiler_params=pltpu.CompilerParams(
            dimension_semantics=("parallel",)),
        interpret=interpret,
    )(x, sig, Wg, bg2, Wrel0, brel02, Wroot0, Wrel1, brel12, Wroot1,
      Wrel2, brel22, Wroot2, W1, b12, W2, b22)


def kernel(x, edge_index, edge_weight, batch, sigma, Wg, bg,
           Wrel0, brel0, Wroot0, Wrel1, brel1, Wroot1, Wrel2, brel2, Wroot2,
           W1, b1, W2, b2):
    return _run(x, sigma.reshape(1, 1), Wg, bg.reshape(1, D),
                Wrel0, brel0.reshape(1, H), Wroot0,
                Wrel1, brel1.reshape(1, H), Wroot1,
                Wrel2, brel2.reshape(1, H), Wroot2,
                W1, b1.reshape(1, H), W2, b2.reshape(1, OUT))
